# baseline (speedup 1.0000x reference)
import functools

import jax
import jax.numpy as jnp
from jax import lax
from jax.experimental import pallas as pl
from jax.experimental.pallas import tpu as pltpu

B, D, M = 256, 256, 100000
TEMP = 0.05
EPS = 0.1
MARGIN = 0.1
MB = 5000
NBLK = M // MB
NEG = -1e9
LOG2E = 1.4426950408889634
C = 20.0 * LOG2E


def _body(tcol_ref, x_in_ref, feat_ref, sf_ref, lab_ref, out_ce_ref,
          out_l2_ref, s_hbm, s_x, s_xc, s_se, s_fs, s_tl, s_mp, s_mn, s_pl,
          s_hp, s_sb, sem_o, sem_i):
    p = pl.program_id(0)
    m = pl.program_id(1)
    slot = lax.rem(m, 2)

    @pl.when((p == 0) & (m == 0))
    def _init():
        x = x_in_ref[...]
        xn = x * lax.rsqrt(jnp.sum(x * x, axis=1, keepdims=True))
        s_x[...] = xn.astype(jnp.bfloat16)
        s_xc[...] = (xn * C).astype(jnp.bfloat16)
        s_se[...] = jnp.zeros((B, 1), jnp.float32)
        s_fs[...] = jnp.zeros((1, D), jnp.float32)
        s_tl[...] = jnp.zeros((B, 1), jnp.float32)
        s_mp[...] = jnp.full((B, 1), NEG, jnp.float32)
        s_mn[...] = jnp.full((B, 1), NEG, jnp.float32)

    x = s_x[...]
    dn = (((1,), (1,)), ((), ()))
    lab = lab_ref[0]
    tcol = tcol_ref[...]
    posm = lab == tcol

    @pl.when(p == 0)
    def _ph0():
        sim = lax.dot_general(x, sf_ref[...].astype(jnp.bfloat16), dn,
                              preferred_element_type=jnp.float32)
        @pl.when(m >= 2)
        def _drain_prev():
            pltpu.make_async_copy(s_sb.at[slot], s_hbm.at[m - 2],
                                  sem_o.at[slot]).wait()
        s_sb[slot] = sim
        pltpu.make_async_copy(s_sb.at[slot], s_hbm.at[m],
                              sem_o.at[slot]).start()
        f = feat_ref[...].astype(jnp.bfloat16)
        rc = lax.dot_general(s_xc[...], f, dn,
                             preferred_element_type=jnp.float32)
        s_se[...] += jnp.sum(jnp.exp2(rc), axis=1, keepdims=True)
        ones = jnp.ones((1, MB), jnp.bfloat16)
        s_fs[...] += lax.dot_general(ones, f, (((1,), (0,)), ((), ())),
                                     preferred_element_type=jnp.float32)
        col = m * MB + lax.broadcasted_iota(jnp.int32, (1, MB), 1)
        s_tl[...] += jnp.sum(jnp.where(col == tcol, rc, 0.0), axis=1,
                             keepdims=True)
        s_mp[...] = jnp.maximum(
            s_mp[...], jnp.max(jnp.where(posm, sim, NEG), axis=1, keepdims=True))
        s_mn[...] = jnp.maximum(
            s_mn[...], jnp.max(jnp.where(posm, NEG, sim), axis=1, keepdims=True))

    @pl.when((p == 1) & (m == 0))
    def _mid():
        s_hp[...] = jnp.where(s_mp[...] > -1e8, 1.0, 0.0)
        s_mn[...] = s_mn[...] + MARGIN
        s_mp[...] = jnp.maximum(0.6, s_mp[...]) - MARGIN
        s_pl[...] = jnp.zeros((B, 1), jnp.float32)
        pltpu.make_async_copy(s_sb.at[0], s_hbm.at[NBLK - 2],
                              sem_o.at[0]).wait()
        pltpu.make_async_copy(s_sb.at[1], s_hbm.at[NBLK - 1],
                              sem_o.at[1]).wait()
        pltpu.make_async_copy(s_hbm.at[0], s_sb.at[0], sem_i.at[0]).start()
        pltpu.make_async_copy(s_hbm.at[1], s_sb.at[1], sem_i.at[1]).start()

    @pl.when(p == 1)
    def _ph1():
        pltpu.make_async_copy(s_hbm.at[m], s_sb.at[slot],
                              sem_i.at[slot]).wait()
        sim = s_sb[slot]
        val = jnp.where(posm,
                        jnp.where(sim < s_mn[...], 1.0 - sim, 0.0),
                        jnp.where(sim > s_mp[...], sim, 0.0))
        s_pl[...] += jnp.sum(val, axis=1, keepdims=True)

        @pl.when(m + 2 < NBLK)
        def _next_fetch():
            pltpu.make_async_copy(s_hbm.at[m + 2], s_sb.at[slot],
                                  sem_i.at[slot]).start()

    @pl.when((p == 1) & (m == NBLK - 1))
    def _fin():
        lse = jnp.log(s_se[...])
        xi = x_in_ref[...]
        xn = xi * lax.rsqrt(jnp.sum(xi * xi, axis=1, keepdims=True))
        so = jnp.sum(xn * s_fs[...], axis=1, keepdims=True) * (1.0 / TEMP)
        tl = s_tl[...] * (20.0 / C)
        ce = ((1.0 - EPS) * (lse - tl) + (EPS / M) * (M * lse - so))
        out_ce_ref[...] = jnp.sum(ce, keepdims=True).reshape(1, 1) / B
        li = jnp.where(s_hp[...] > 0, s_pl[...], 0.0)
        out_l2_ref[...] = jnp.sum(li, keepdims=True).reshape(1, 1) / B


@functools.partial(jax.jit, static_argnames=("interpret",))
def _run(inputs, targets, features, sample_features, sample_labels,
         interpret=False):
    tcol = targets.reshape(B, 1)
    lab3 = sample_labels.reshape(NBLK, 1, MB)
    f32 = jnp.float32
    out_ce, out_l2, _sim_spill = pl.pallas_call(
        _body,
        grid=(2, NBLK),
        in_specs=[
            pl.BlockSpec((B, 1), lambda p, m: (0, 0)),
            pl.BlockSpec((B, D), lambda p, m: (0, 0)),
            pl.BlockSpec((MB, D), lambda p, m: (m * (1 - p), 0)),
            pl.BlockSpec((MB, D), lambda p, m: (m * (1 - p), 0)),
            pl.BlockSpec((1, 1, MB), lambda p, m: (m, 0, 0)),
        ],
        out_specs=[
            pl.BlockSpec((1, 1), lambda p, m: (0, 0)),
            pl.BlockSpec((1, 1), lambda p, m: (0, 0)),
            pl.BlockSpec(memory_space=pl.ANY),
        ],
        out_shape=[
            jax.ShapeDtypeStruct((1, 1), f32),
            jax.ShapeDtypeStruct((1, 1), f32),
            jax.ShapeDtypeStruct((NBLK, B, MB), f32),
        ],
        scratch_shapes=[
            pltpu.VMEM((B, D), jnp.bfloat16), pltpu.VMEM((B, D), jnp.bfloat16),
            pltpu.VMEM((B, 1), f32), pltpu.VMEM((1, D), f32),
            pltpu.VMEM((B, 1), f32), pltpu.VMEM((B, 1), f32),
            pltpu.VMEM((B, 1), f32), pltpu.VMEM((B, 1), f32),
            pltpu.VMEM((B, 1), f32),
            pltpu.VMEM((2, B, MB), f32),
            pltpu.SemaphoreType.DMA((2,)), pltpu.SemaphoreType.DMA((2,)),
        ],
        interpret=interpret,
    )(tcol, inputs, features, sample_features, lab3)
    return out_ce[0, 0], out_l2[0, 0]


def kernel(inputs, targets, features, sample_features, sample_labels):
    return _run(inputs, targets, features, sample_features, sample_labels)


# speedup vs baseline: 1.1270x; 1.1270x over previous
import functools

import jax
import jax.numpy as jnp
from jax import lax
from jax.experimental import pallas as pl
from jax.experimental.pallas import tpu as pltpu

B, D, M = 256, 256, 100000
TEMP = 0.05
EPS = 0.1
MARGIN = 0.1
MB = 5000
NBLK = M // MB
NEG = -1e9
LOG2E = 1.4426950408889634
C = 20.0 * LOG2E


def _body(tcol_ref, x_in_ref, feat_ref, sf_ref, lab_ref, out_ce_ref,
          out_l2_ref, s_x, s_xc, s_se, s_fs, s_tl, s_mp, s_mn, s_pl, s_hp):
    p = pl.program_id(0)
    m = pl.program_id(1)

    @pl.when((p == 0) & (m == 0))
    def _init():
        x = x_in_ref[...]
        xn = x * lax.rsqrt(jnp.sum(x * x, axis=1, keepdims=True))
        s_x[...] = xn.astype(jnp.bfloat16)
        s_xc[...] = (xn * C).astype(jnp.bfloat16)
        s_se[...] = jnp.zeros((B, 1), jnp.float32)
        s_fs[...] = jnp.zeros((1, D), jnp.float32)
        s_tl[...] = jnp.zeros((B, 1), jnp.float32)
        s_mp[...] = jnp.full((B, 1), NEG, jnp.float32)
        s_mn[...] = jnp.full((B, 1), NEG, jnp.float32)

    x = s_x[...]
    dn = (((1,), (1,)), ((), ()))
    sim = lax.dot_general(x, sf_ref[...].astype(jnp.bfloat16), dn,
                          preferred_element_type=jnp.float32)
    lab = lab_ref[0]
    tcol = tcol_ref[...]
    posm = lab == tcol

    @pl.when(p == 0)
    def _ph0():
        f = feat_ref[...].astype(jnp.bfloat16)
        rc = lax.dot_general(s_xc[...], f, dn,
                             preferred_element_type=jnp.float32)
        s_se[...] += jnp.sum(jnp.exp2(rc), axis=1, keepdims=True)
        ones = jnp.ones((1, MB), jnp.bfloat16)
        s_fs[...] += lax.dot_general(ones, f, (((1,), (0,)), ((), ())),
                                     preferred_element_type=jnp.float32)
        col = m * MB + lax.broadcasted_iota(jnp.int32, (1, MB), 1)
        s_tl[...] += jnp.sum(jnp.where(col == tcol, rc, 0.0), axis=1,
                             keepdims=True)
        s_mp[...] = jnp.maximum(
            s_mp[...], jnp.max(jnp.where(posm, sim, NEG), axis=1, keepdims=True))
        s_mn[...] = jnp.maximum(
            s_mn[...], jnp.max(jnp.where(posm, NEG, sim), axis=1, keepdims=True))

    @pl.when((p == 1) & (m == 0))
    def _mid():
        s_hp[...] = jnp.where(s_mp[...] > -1e8, 1.0, 0.0)
        s_mn[...] = s_mn[...] + MARGIN
        s_mp[...] = jnp.maximum(0.6, s_mp[...]) - MARGIN
        s_pl[...] = jnp.zeros((B, 1), jnp.float32)

    @pl.when(p == 1)
    def _ph1():
        val = jnp.where(posm,
                        jnp.where(sim < s_mn[...], 1.0 - sim, 0.0),
                        jnp.where(sim > s_mp[...], sim, 0.0))
        s_pl[...] += jnp.sum(val, axis=1, keepdims=True)

    @pl.when((p == 1) & (m == NBLK - 1))
    def _fin():
        lse = jnp.log(s_se[...])
        xi = x_in_ref[...]
        xn = xi * lax.rsqrt(jnp.sum(xi * xi, axis=1, keepdims=True))
        so = jnp.sum(xn * s_fs[...], axis=1, keepdims=True) * (1.0 / TEMP)
        tl = s_tl[...] * (20.0 / C)
        ce = ((1.0 - EPS) * (lse - tl) + (EPS / M) * (M * lse - so))
        out_ce_ref[...] = jnp.sum(ce, keepdims=True).reshape(1, 1) / B
        li = jnp.where(s_hp[...] > 0, s_pl[...], 0.0)
        out_l2_ref[...] = jnp.sum(li, keepdims=True).reshape(1, 1) / B


@functools.partial(jax.jit, static_argnames=("interpret",))
def _run(inputs, targets, features, sample_features, sample_labels,
         interpret=False):
    tcol = targets.reshape(B, 1)
    lab3 = sample_labels.reshape(NBLK, 1, MB)
    f32 = jnp.float32
    out_ce, out_l2 = pl.pallas_call(
        _body,
        grid=(2, NBLK),
        in_specs=[
            pl.BlockSpec((B, 1), lambda p, m: (0, 0)),
            pl.BlockSpec((B, D), lambda p, m: (0, 0)),
            pl.BlockSpec((MB, D), lambda p, m: (m * (1 - p), 0)),
            pl.BlockSpec((MB, D), lambda p, m: (m, 0)),
            pl.BlockSpec((1, 1, MB), lambda p, m: (m, 0, 0)),
        ],
        out_specs=[
            pl.BlockSpec((1, 1), lambda p, m: (0, 0)),
            pl.BlockSpec((1, 1), lambda p, m: (0, 0)),
        ],
        out_shape=[
            jax.ShapeDtypeStruct((1, 1), f32),
            jax.ShapeDtypeStruct((1, 1), f32),
        ],
        scratch_shapes=[
            pltpu.VMEM((B, D), jnp.bfloat16), pltpu.VMEM((B, D), jnp.bfloat16),
            pltpu.VMEM((B, 1), f32), pltpu.VMEM((1, D), f32),
            pltpu.VMEM((B, 1), f32), pltpu.VMEM((B, 1), f32),
            pltpu.VMEM((B, 1), f32), pltpu.VMEM((B, 1), f32),
            pltpu.VMEM((B, 1), f32),
        ],
        interpret=interpret,
    )(tcol, inputs, features, sample_features, lab3)
    return out_ce[0, 0], out_l2[0, 0]


def kernel(inputs, targets, features, sample_features, sample_labels):
    return _run(inputs, targets, features, sample_features, sample_labels)
